# baseline (speedup 1.0000x reference)
import functools

import jax
import jax.numpy as jnp
import numpy as np
from jax import lax
from jax.experimental import pallas as pl
from jax.experimental.pallas import tpu as pltpu
from jax.experimental.pallas import tpu_sc as plsc


def _pos_encoding(max_seq_len, d_model):
    pos = np.arange(max_seq_len, dtype=np.float64)[:, None]
    i = np.arange(d_model, dtype=np.float64)[None, :]
    angle = pos / np.power(10000.0, (2.0 * (np.floor(i / 2.0))) / d_model)
    pe = np.where((np.arange(d_model)[None, :] % 2) == 0, np.sin(angle), np.cos(angle))
    return pe.astype(np.float32)


_NW = 32
_SUB = ((0, 128), (128, 72))


@functools.partial(jax.jit, static_argnames=("b", "s", "d"))
def _emb_lookup(tokens_flat, table, pe, *, b, s, d):
    n_rows = b * s
    per_w = n_rows // _NW
    n_chunks = per_w // s
    b_per_w = b // _NW
    mesh = plsc.VectorSubcoreMesh(core_axis_name="c", subcore_axis_name="s")

    @functools.partial(
        pl.kernel,
        out_type=jax.ShapeDtypeStruct((b, s, d), jnp.float32),
        mesh=mesh,
        scratch_types=[
            pltpu.VMEM((per_w,), jnp.int32),
            pltpu.VMEM((s * d,), jnp.float32),
            pltpu.VMEM((2, s, d), jnp.float32),
            pltpu.SemaphoreType.DMA,
            pltpu.SemaphoreType.DMA,
        ],
        compiler_params=pltpu.CompilerParams(use_tc_tiling_on_sc=False),
    )
    def k(tokens_hbm, table_hbm, pe_hbm, out_hbm, idx_v, pe_v, gbuf, gsem, osem):
        wid = lax.axis_index("s") * 2 + lax.axis_index("c")
        base = wid * per_w
        bbase = wid * b_per_w
        pltpu.sync_copy(tokens_hbm.at[pl.ds(base, per_w)], idx_v)
        pltpu.sync_copy(pe_hbm, pe_v)

        def gathers(c, buf):
            off = c * s
            for so, n in _SUB:
                pltpu.async_copy(
                    table_hbm.at[idx_v.at[pl.ds(off + so, n)]],
                    gbuf.at[buf, pl.ds(so, n)],
                    gsem,
                )

        def wait_gathers(buf):
            for so, n in _SUB:
                pltpu.make_async_copy(
                    table_hbm.at[idx_v.at[pl.ds(so, n)]],
                    gbuf.at[buf, pl.ds(so, n)],
                    gsem,
                ).wait()

        def out_copy(c, buf):
            pltpu.async_copy(gbuf.at[buf], out_hbm.at[bbase + c], osem)

        def wait_out(buf):
            pltpu.make_async_copy(gbuf.at[buf], out_hbm.at[bbase], osem).wait()

        def add_pe(buf):
            def body(r, carry):
                for j in range(d // 16):
                    sl = pl.ds(j * 16, 16)
                    gbuf[buf, r, sl] = gbuf[buf, r, sl] + pe_v[pl.ds(r * d + j * 16, 16)]
                return carry

            lax.fori_loop(0, s, body, 0, unroll=2)

        gathers(0, 0)

        def chunk_body(c, carry):
            for buf in range(2):
                c2 = 2 * c + buf
                nb = 1 - buf

                @pl.when(c2 + 1 < n_chunks)
                def _():
                    @pl.when(c2 >= 1)
                    def _():
                        wait_out(nb)

                    gathers(c2 + 1, nb)

                wait_gathers(buf)
                add_pe(buf)
                out_copy(c2, buf)
            return carry

        lax.fori_loop(0, n_chunks // 2, chunk_body, 0)
        wait_out(0)
        wait_out(1)

    return k(tokens_flat, table, pe)


def kernel(tokens, table):
    b, s = tokens.shape
    v, d = table.shape
    pe_flat = jnp.asarray(_pos_encoding(s, d).reshape(-1))
    tokens_flat = tokens.reshape(-1).astype(jnp.int32)
    return _emb_lookup(tokens_flat, table, pe_flat, b=b, s=s, d=d)


# speedup vs baseline: 1.0227x; 1.0227x over previous
import functools

import jax
import jax.numpy as jnp
import numpy as np
from jax import lax
from jax.experimental import pallas as pl
from jax.experimental.pallas import tpu as pltpu
from jax.experimental.pallas import tpu_sc as plsc


def _pos_encoding(max_seq_len, d_model):
    pos = np.arange(max_seq_len, dtype=np.float64)[:, None]
    i = np.arange(d_model, dtype=np.float64)[None, :]
    angle = pos / np.power(10000.0, (2.0 * (np.floor(i / 2.0))) / d_model)
    pe = np.where((np.arange(d_model)[None, :] % 2) == 0, np.sin(angle), np.cos(angle))
    return pe.astype(np.float32)


_NW = 32
_CHUNK = 256


@functools.partial(jax.jit, static_argnames=("b", "s", "d"))
def _emb_lookup(tokens2d, table, pe_flat, *, b, s, d):
    n_rows = b * s
    pe_n = s * d
    per_w = n_rows // _NW
    irows_w = per_w // 128
    n_chunks = per_w // _CHUNK
    orows_c = _CHUNK * d // 128
    mesh = plsc.VectorSubcoreMesh(core_axis_name="c", subcore_axis_name="s")

    @functools.partial(
        pl.kernel,
        out_type=jax.ShapeDtypeStruct((n_rows * d // 128, 128), jnp.float32),
        mesh=mesh,
        scratch_types=[
            pltpu.VMEM((irows_w, 128), jnp.int32),
            pltpu.VMEM((pe_n,), jnp.float32),
            pltpu.VMEM((2, _CHUNK, d), jnp.float32),
            pltpu.VMEM((2, orows_c, 128), jnp.float32),
            pltpu.SemaphoreType.DMA,
            pltpu.SemaphoreType.DMA,
        ],
        compiler_params=pltpu.CompilerParams(use_tc_tiling_on_sc=False),
    )
    def k(tokens_hbm, table_hbm, pe_hbm, out_hbm, idx_v, pe_v, gbuf, obuf, gsem, osem):
        wid = lax.axis_index("s") * 2 + lax.axis_index("c")
        obase = wid * (per_w * d // 128)
        pltpu.sync_copy(tokens_hbm.at[pl.ds(wid * irows_w, irows_w)], idx_v)
        pltpu.sync_copy(pe_hbm, pe_v)

        def gathers(c, buf):
            for j in range(_CHUNK // 128):
                pltpu.async_copy(
                    table_hbm.at[idx_v.at[c * (_CHUNK // 128) + j]],
                    gbuf.at[buf, pl.ds(j * 128, 128)],
                    gsem,
                )

        def wait_gathers(buf):
            for j in range(_CHUNK // 128):
                pltpu.make_async_copy(
                    table_hbm.at[idx_v.at[j]],
                    gbuf.at[buf, pl.ds(j * 128, 128)],
                    gsem,
                ).wait()

        def out_copy(c, buf):
            pltpu.async_copy(
                obuf.at[buf], out_hbm.at[pl.ds(obase + c * orows_c, orows_c)], osem
            )

        def wait_out(buf):
            pltpu.make_async_copy(
                obuf.at[buf], out_hbm.at[pl.ds(obase, orows_c)], osem
            ).wait()

        def add_pe(c2, buf):
            p0 = lax.rem(c2 * (_CHUNK * d), pe_n)

            def body(r, p):
                for j in range(d // 16):
                    sl = pl.ds(j * 16, 16)
                    v = gbuf[buf, r, sl] + pe_v[pl.ds(p + j * 16, 16)]
                    obuf[buf, r // 2, pl.ds((r % 2) * d + j * 16, 16)] = v
                pn = p + d
                return lax.select(pn >= pe_n, pn - pe_n, pn)

            lax.fori_loop(0, _CHUNK, body, p0, unroll=2)

        gathers(0, 0)

        def chunk_body(c, carry):
            for buf in range(2):
                c2 = 2 * c + buf
                nb = 1 - buf

                @pl.when(c2 + 1 < n_chunks)
                def _():
                    gathers(c2 + 1, nb)

                wait_gathers(buf)

                @pl.when(c2 >= 2)
                def _():
                    wait_out(buf)

                add_pe(c2, buf)
                out_copy(c2, buf)
            return carry

        lax.fori_loop(0, n_chunks // 2, chunk_body, 0)
        wait_out(0)
        wait_out(1)

    return k(tokens2d, table, pe_flat)


def kernel(tokens, table):
    b, s = tokens.shape
    v, d = table.shape
    pe_flat = jnp.asarray(_pos_encoding(s, d).reshape(-1))
    tokens2d = tokens.reshape(-1, 128).astype(jnp.int32)
    out = _emb_lookup(tokens2d, table, pe_flat, b=b, s=s, d=d)
    return out.reshape(b, s, d)


# speedup vs baseline: 1.1741x; 1.1481x over previous
import functools

import jax
import jax.numpy as jnp
import numpy as np
from jax import lax
from jax.experimental import pallas as pl
from jax.experimental.pallas import tpu as pltpu
from jax.experimental.pallas import tpu_sc as plsc


def _pos_encoding(max_seq_len, d_model):
    pos = np.arange(max_seq_len, dtype=np.float64)[:, None]
    i = np.arange(d_model, dtype=np.float64)[None, :]
    angle = pos / np.power(10000.0, (2.0 * (np.floor(i / 2.0))) / d_model)
    pe = np.where((np.arange(d_model)[None, :] % 2) == 0, np.sin(angle), np.cos(angle))
    return pe.astype(np.float32)


_NW = 32
_CHUNK = 200
_RING = 4
_SUB = ((0, 128), (128, 72))


@functools.partial(jax.jit, static_argnames=("n_rows", "d"))
def _emb_lookup(tokens_flat, table, pe, *, n_rows, d):
    per_w = n_rows // _NW
    n_chunks = per_w // _CHUNK
    mesh = plsc.VectorSubcoreMesh(core_axis_name="c", subcore_axis_name="s")

    @functools.partial(
        pl.kernel,
        out_type=jax.ShapeDtypeStruct((n_rows, d), jnp.float32),
        mesh=mesh,
        scratch_types=[
            pltpu.VMEM((per_w,), jnp.int32),
            pltpu.VMEM((_CHUNK, d), jnp.float32),
            pltpu.VMEM((_RING, _CHUNK, d), jnp.float32),
            pltpu.SemaphoreType.DMA,
            pltpu.SemaphoreType.DMA,
        ],
        compiler_params=pltpu.CompilerParams(use_tc_tiling_on_sc=False),
    )
    def k(tokens_hbm, table_hbm, pe_hbm, out_hbm, idx_v, pe_v, gbuf, gsem, osem):
        wid = lax.axis_index("s") * 2 + lax.axis_index("c")
        base = wid * per_w
        pltpu.sync_copy(tokens_hbm.at[pl.ds(base, per_w)], idx_v)
        pltpu.sync_copy(pe_hbm, pe_v)

        def gathers(c, slot):
            off = c * _CHUNK
            for so, n in _SUB:
                pltpu.async_copy(
                    table_hbm.at[idx_v.at[pl.ds(off + so, n)]],
                    gbuf.at[slot, pl.ds(so, n)],
                    gsem,
                )

        def wait_gathers(slot):
            for so, n in _SUB:
                pltpu.make_async_copy(
                    table_hbm.at[idx_v.at[pl.ds(so, n)]],
                    gbuf.at[slot, pl.ds(so, n)],
                    gsem,
                ).wait()

        def out_copy(c, slot):
            pltpu.async_copy(
                gbuf.at[slot], out_hbm.at[pl.ds(base + c * _CHUNK, _CHUNK)], osem
            )

        def wait_out(slot):
            pltpu.make_async_copy(
                gbuf.at[slot], out_hbm.at[pl.ds(base, _CHUNK)], osem
            ).wait()

        def add_pe(slot):
            def body(r, carry):
                for j in range(d // 16):
                    sl = pl.ds(j * 16, 16)
                    gbuf[slot, r, sl] = gbuf[slot, r, sl] + pe_v[r, sl]
                return carry

            lax.fori_loop(0, _CHUNK, body, 0, unroll=2)

        for pre in range(_RING - 1):
            gathers(pre, pre)

        def chunk_body(c, carry):
            for slot in range(_RING):
                c2 = _RING * c + slot
                nslot = (slot + _RING - 1) % _RING

                wait_gathers(slot)
                add_pe(slot)
                out_copy(c2, slot)

                @pl.when(c2 + _RING - 1 < n_chunks)
                def _():
                    @pl.when(c2 >= 1)
                    def _():
                        wait_out(nslot)

                    gathers(c2 + _RING - 1, nslot)

            return carry

        lax.fori_loop(0, n_chunks // _RING, chunk_body, 0)
        for fslot in range(_RING):
            wait_out(fslot)

    return k(tokens_flat, table, pe)


def kernel(tokens, table):
    b, s = tokens.shape
    v, d = table.shape
    n_rows = b * s
    pe = jnp.asarray(_pos_encoding(s, d))
    tokens_flat = tokens.reshape(-1).astype(jnp.int32)
    out = _emb_lookup(tokens_flat, table, pe, n_rows=n_rows, d=d)
    return out.reshape(b, s, d)


# speedup vs baseline: 1.1846x; 1.0090x over previous
import functools

import jax
import jax.numpy as jnp
import numpy as np
from jax import lax
from jax.experimental import pallas as pl
from jax.experimental.pallas import tpu as pltpu
from jax.experimental.pallas import tpu_sc as plsc


def _pos_encoding(max_seq_len, d_model):
    pos = np.arange(max_seq_len, dtype=np.float64)[:, None]
    i = np.arange(d_model, dtype=np.float64)[None, :]
    angle = pos / np.power(10000.0, (2.0 * (np.floor(i / 2.0))) / d_model)
    pe = np.where((np.arange(d_model)[None, :] % 2) == 0, np.sin(angle), np.cos(angle))
    return pe.astype(np.float32)


_NW = 32
_CHUNK = 200
_RING = 4
_SUB = ((0, 128), (128, 72))
_P = 2


@functools.partial(jax.jit, static_argnames=("n_rows", "d"))
def _emb_lookup(tokens_flat, table, pe, *, n_rows, d):
    per_w = n_rows // _NW
    n_chunks = per_w // _CHUNK
    mesh = plsc.VectorSubcoreMesh(core_axis_name="c", subcore_axis_name="s")

    @functools.partial(
        pl.kernel,
        out_type=jax.ShapeDtypeStruct((n_rows, d), jnp.float32),
        mesh=mesh,
        scratch_types=[
            pltpu.VMEM((per_w,), jnp.int32),
            pltpu.VMEM((_CHUNK, d), jnp.float32),
            pltpu.VMEM((_RING, _CHUNK, d), jnp.float32),
            pltpu.SemaphoreType.DMA,
            pltpu.SemaphoreType.DMA,
        ],
        compiler_params=pltpu.CompilerParams(use_tc_tiling_on_sc=False),
    )
    def k(tokens_hbm, table_hbm, pe_hbm, out_hbm, idx_v, pe_v, gbuf, gsem, osem):
        wid = lax.axis_index("s") * 2 + lax.axis_index("c")
        base = wid * per_w
        pltpu.sync_copy(tokens_hbm.at[pl.ds(base, per_w)], idx_v)
        pltpu.sync_copy(pe_hbm, pe_v)

        def gathers(c, slot):
            off = c * _CHUNK
            for so, n in _SUB:
                pltpu.async_copy(
                    table_hbm.at[idx_v.at[pl.ds(off + so, n)]],
                    gbuf.at[slot, pl.ds(so, n)],
                    gsem,
                )

        def wait_gathers(slot):
            for so, n in _SUB:
                pltpu.make_async_copy(
                    table_hbm.at[idx_v.at[pl.ds(so, n)]],
                    gbuf.at[slot, pl.ds(so, n)],
                    gsem,
                ).wait()

        def out_copy(c, slot):
            pltpu.async_copy(
                gbuf.at[slot], out_hbm.at[pl.ds(base + c * _CHUNK, _CHUNK)], osem
            )

        def wait_out(slot):
            pltpu.make_async_copy(
                gbuf.at[slot], out_hbm.at[pl.ds(base, _CHUNK)], osem
            ).wait()

        def add_pe(slot):
            def body(r, carry):
                for j in range(d // 16):
                    sl = pl.ds(j * 16, 16)
                    gbuf[slot, r, sl] = gbuf[slot, r, sl] + pe_v[r, sl]
                return carry

            lax.fori_loop(0, _CHUNK, body, 0, unroll=2)

        for pre in range(_RING - 1):
            gathers(pre, pre)

        def chunk_body(c, carry):
            for slot in range(_RING):
                c2 = _RING * c + slot
                nslot = (slot + _RING - 1) % _RING

                wait_gathers(slot)
                add_pe(slot)
                out_copy(c2, slot)

                @pl.when(c2 + _RING - 1 < n_chunks)
                def _():
                    @pl.when(c2 >= 1)
                    def _():
                        wait_out(nslot)

                    gathers(c2 + _RING - 1, nslot)

            return carry

        lax.fori_loop(0, n_chunks // _RING, chunk_body, 0)
        for fslot in range(_RING):
            wait_out(fslot)

    return k(tokens_flat, table, pe)


def kernel(tokens, table):
    b, s = tokens.shape
    v, d = table.shape
    bp = b // _P
    pe = jnp.asarray(_pos_encoding(s, d))
    pieces = []
    for p in range(_P):
        tok_p = tokens[p * bp:(p + 1) * bp].reshape(-1).astype(jnp.int32)
        out_p = _emb_lookup(tok_p, table, pe, n_rows=bp * s, d=d)
        pieces.append(out_p.reshape(bp, s, d))
    return jnp.concatenate(pieces, axis=0)
